# Initial kernel scaffold; baseline (speedup 1.0000x reference)
import jax
import jax.numpy as jnp
from jax.experimental import pallas as pl


def kernel(z, W_pre, b_pre, tables):
    raise NotImplementedError("write your pallas kernel here")


# speedup vs baseline: 1.2791x; 1.2791x over previous
import functools

import jax
import jax.numpy as jnp
from jax import lax
from jax.experimental import pallas as pl
from jax.experimental.pallas import tpu as pltpu
from jax.experimental.pallas import tpu_sc as plsc

VOCAB = 8192
EMBED = 256
T = 4
BK = 2048
VB = 1024
NVB = VOCAB // VB

_info = plsc.get_sparse_core_info()
_NC, _NS = _info.num_cores, _info.num_subcores
NW = _NC * _NS
ROWS = BK * T
RPW = ROWS // NW
CHUNK = 128
NCH = RPW // CHUNK


def _pre_body(z_ref, w_ref, b_ref, out_ref):
    out_ref[...] = jnp.dot(z_ref[...], w_ref[...]) + b_ref[...]


def _dist_body(zp_ref, z2_ref, e2_ref, tab_ref, tok_ref, gidx_ref,
               best_ref, bi_ref):
    t = pl.program_id(0)
    vb = pl.program_id(1)

    @pl.when(vb == 0)
    def _init():
        best_ref[...] = jnp.full_like(best_ref, jnp.inf)
        bi_ref[...] = jnp.zeros_like(bi_ref)

    zp = zp_ref[0]
    e = tab_ref[0]
    m = lax.dot_general(zp, e, (((1,), (1,)), ((), ())))
    d = (z2_ref[0] + e2_ref[0]) - 2.0 * m
    lmin = jnp.min(d, axis=1, keepdims=True)
    iota = lax.broadcasted_iota(jnp.int32, d.shape, 1)
    larg = jnp.min(jnp.where(d == lmin, iota, VOCAB),
                   axis=1, keepdims=True) + vb * VB
    better = lmin < best_ref[...]
    best_ref[...] = jnp.where(better, lmin, best_ref[...])
    bi_ref[...] = jnp.where(better, larg, bi_ref[...])

    @pl.when(vb == NVB - 1)
    def _fin():
        tok = bi_ref[...]
        tok_ref[0] = tok
        gidx_ref[0] = tok + t * VOCAB


def _distance_argmin(zp_t, z2_t, e2_t, tables):
    return pl.pallas_call(
        _dist_body,
        grid=(T, NVB),
        in_specs=[
            pl.BlockSpec((1, BK, EMBED), lambda t, v: (t, 0, 0)),
            pl.BlockSpec((1, BK, 1), lambda t, v: (t, 0, 0)),
            pl.BlockSpec((1, 1, VB), lambda t, v: (t, 0, v)),
            pl.BlockSpec((1, VB, EMBED), lambda t, v: (t, v, 0)),
        ],
        out_specs=[
            pl.BlockSpec((1, BK, 1), lambda t, v: (t, 0, 0)),
            pl.BlockSpec((1, BK, 1), lambda t, v: (t, 0, 0)),
        ],
        out_shape=[
            jax.ShapeDtypeStruct((T, BK, 1), jnp.int32),
            jax.ShapeDtypeStruct((T, BK, 1), jnp.int32),
        ],
        scratch_shapes=[
            pltpu.VMEM((BK, 1), jnp.float32),
            pltpu.VMEM((BK, 1), jnp.int32),
        ],
    )(zp_t, z2_t, e2_t, tables)


@functools.partial(
    pl.kernel,
    mesh=plsc.VectorSubcoreMesh(core_axis_name="c", subcore_axis_name="s"),
    out_type=jax.ShapeDtypeStruct((NW * NCH, CHUNK, EMBED), jnp.float32),
    scratch_types=[
        pltpu.VMEM((NCH, CHUNK), jnp.int32),
        pltpu.VMEM((NCH, CHUNK, EMBED), jnp.float32),
        pltpu.SemaphoreType.DMA,
    ],
)
def _sc_gather(tab_hbm, idx_hbm, out_hbm, idx_v, rows_v, sem):
    wid = lax.axis_index("s") * _NC + lax.axis_index("c")
    pltpu.sync_copy(idx_hbm.at[pl.ds(wid * NCH, NCH)], idx_v)
    cps = [
        pltpu.async_copy(tab_hbm.at[idx_v.at[j]], rows_v.at[j], sem)
        for j in range(NCH)
    ]
    for cp in cps:
        cp.wait()
    pltpu.sync_copy(rows_v, out_hbm.at[pl.ds(wid * NCH, NCH)])


def kernel(z, W_pre, b_pre, tables):
    n, feat = z.shape
    zp = pl.pallas_call(
        _pre_body,
        out_shape=jax.ShapeDtypeStruct((n, EMBED), jnp.float32),
    )(z, W_pre, b_pre.reshape(1, EMBED))

    z2 = jnp.sum(zp ** 2, axis=1)
    e2 = jnp.sum(tables ** 2, axis=-1)

    zp_t = zp.reshape(BK, T, EMBED).transpose(1, 0, 2)
    z2_t = z2.reshape(BK, T).T.reshape(T, BK, 1)
    e2_t = e2.reshape(T, 1, VOCAB)

    tok4, gidx4 = _distance_argmin(zp_t, z2_t, e2_t, tables)

    tokens = tok4.reshape(T, BK).T.reshape(ROWS)
    gidx = gidx4.reshape(T, BK).T.reshape(NW * NCH, CHUNK)

    z_q = _sc_gather(tables.reshape(T * VOCAB, EMBED), gidx)
    return (tokens, z_q.reshape(ROWS, EMBED))
